# baseline (speedup 1.0000x reference)
import functools

import jax
import jax.numpy as jnp
from jax.experimental import pallas as pl
from jax.experimental.pallas import tpu as pltpu

_N = 4096
_R = 256
_BIG = 1e10


def _chamfer_l1_kernel(predT_ref, targetT_ref, pointsT_ref, maskT_ref,
                       targetN_ref, pointsN_ref, maskN_ref, out_ref,
                       acc_ref, *, n_batch):
    b = pl.program_id(0)

    @pl.when(b == 0)
    def _init():
        acc_ref[0] = 0.0
        acc_ref[1] = 0.0
        acc_ref[2] = 0.0

    p = predT_ref[0]
    t = targetT_ref[0]
    pts = pointsT_ref[0]
    m = maskT_ref[0]

    cnt = jnp.sum(m)
    l1_num = jnp.sum(jnp.abs(p - t) * m) * (1.0 / 3.0)

    bpt = pts + p
    bx = bpt[0:1, :]
    by = bpt[1:2, :]
    bz = bpt[2:3, :]
    wcol = (1.0 - m) * _BIG

    def body(i, carry):
        sum_ab, colmin = carry
        r0 = i * _R
        a_blk = (pointsN_ref[0, pl.ds(r0, _R), :]
                 + targetN_ref[0, pl.ds(r0, _R), :])
        m_blk = maskN_ref[0, pl.ds(r0, _R), :]
        ax = a_blk[:, 0:1]
        ay = a_blk[:, 1:2]
        az = a_blk[:, 2:3]
        dx = ax - bx
        dy = ay - by
        dz = az - bz
        dist = dx * dx + dy * dy + dz * dz
        min_ab = jnp.min(dist + wcol, axis=1, keepdims=True)
        sum_ab = sum_ab + jnp.sum(jnp.maximum(min_ab, 0.0) * m_blk)
        wrow = (1.0 - m_blk) * _BIG
        colmin = jnp.minimum(colmin, jnp.min(dist + wrow, axis=0,
                                             keepdims=True))
        return sum_ab, colmin

    sum_ab, colmin = jax.lax.fori_loop(
        0, _N // _R, body,
        (jnp.float32(0.0), jnp.full((1, _N), _BIG, jnp.float32)))
    sum_ba = jnp.sum(jnp.maximum(colmin, 0.0) * m)
    cd_b = (sum_ab + sum_ba) / cnt

    acc_ref[0] = acc_ref[0] + l1_num
    acc_ref[1] = acc_ref[1] + cnt
    acc_ref[2] = acc_ref[2] + cd_b

    @pl.when(b == n_batch - 1)
    def _emit():
        l1 = acc_ref[0] / acc_ref[1]
        cd = acc_ref[2] * (1.0 / n_batch)
        out_ref[0, 0] = 0.5 * (l1 + cd)


@jax.jit
def kernel(pred, target, mask, points):
    B, N, D = pred.shape
    predT = jnp.transpose(pred, (0, 2, 1))
    targetT = jnp.transpose(target, (0, 2, 1))
    pointsT = jnp.transpose(points, (0, 2, 1))
    maskf = mask.astype(jnp.float32)
    maskT = maskf.reshape(B, 1, N)
    maskN = maskf.reshape(B, N, 1)

    out = pl.pallas_call(
        functools.partial(_chamfer_l1_kernel, n_batch=B),
        grid=(B,),
        in_specs=[
            pl.BlockSpec((1, D, N), lambda b: (b, 0, 0)),
            pl.BlockSpec((1, D, N), lambda b: (b, 0, 0)),
            pl.BlockSpec((1, D, N), lambda b: (b, 0, 0)),
            pl.BlockSpec((1, 1, N), lambda b: (b, 0, 0)),
            pl.BlockSpec((1, N, D), lambda b: (b, 0, 0)),
            pl.BlockSpec((1, N, D), lambda b: (b, 0, 0)),
            pl.BlockSpec((1, N, 1), lambda b: (b, 0, 0)),
        ],
        out_specs=pl.BlockSpec((1, 1), lambda b: (0, 0),
                               memory_space=pltpu.SMEM),
        out_shape=jax.ShapeDtypeStruct((1, 1), jnp.float32),
        scratch_shapes=[pltpu.SMEM((4,), jnp.float32)],
    )(predT, targetT, pointsT, maskT, target, points, maskN)
    return out[0, 0]


# speedup vs baseline: 1.3226x; 1.3226x over previous
import functools

import jax
import jax.numpy as jnp
from jax.experimental import pallas as pl
from jax.experimental.pallas import tpu as pltpu

_N = 4096
_R = 256
_BIG = 1e10


def _chamfer_l1_kernel(pred8T_ref, target8T_ref, points8T_ref, maskT_ref,
                       target8N_ref, points8N_ref, maskN_ref, out_ref,
                       acc_ref, *, n_batch):
    b = pl.program_id(0)

    @pl.when(b == 0)
    def _init():
        acc_ref[0] = 0.0
        acc_ref[1] = 0.0
        acc_ref[2] = 0.0

    p = pred8T_ref[0]
    t = target8T_ref[0]
    pts = points8T_ref[0]
    m = maskT_ref[0]

    cnt = jnp.sum(m)
    l1_num = jnp.sum(jnp.abs(p - t) * m) * (1.0 / 3.0)

    bpt = pts + p
    b2 = jnp.sum(bpt * bpt, axis=0, keepdims=True)
    bneg = -2.0 * bpt
    cb = b2 + (1.0 - m) * _BIG

    def body(i, carry):
        sum_ab, colmin = carry
        r0 = i * _R
        a_blk = (points8N_ref[0, pl.ds(r0, _R), :]
                 + target8N_ref[0, pl.ds(r0, _R), :])
        m_blk = maskN_ref[0, pl.ds(r0, _R), :]
        a2 = jnp.sum(a_blk * a_blk, axis=1, keepdims=True)
        ca = a2 + (1.0 - m_blk) * _BIG
        s = jax.lax.dot_general(a_blk, bneg, (((1,), (0,)), ((), ())),
                                preferred_element_type=jnp.float32)
        rv = jnp.min(s + cb, axis=1, keepdims=True)
        sum_ab = sum_ab + jnp.sum(jnp.maximum(rv + a2, 0.0) * m_blk)
        cv = jnp.min(s + ca, axis=0, keepdims=True)
        colmin = jnp.minimum(colmin, cv)
        return sum_ab, colmin

    sum_ab, colmin = jax.lax.fori_loop(
        0, _N // _R, body,
        (jnp.float32(0.0), jnp.full((1, _N), _BIG, jnp.float32)))
    sum_ba = jnp.sum(jnp.maximum(colmin + b2, 0.0) * m)
    cd_b = (sum_ab + sum_ba) / cnt

    acc_ref[0] = acc_ref[0] + l1_num
    acc_ref[1] = acc_ref[1] + cnt
    acc_ref[2] = acc_ref[2] + cd_b

    @pl.when(b == n_batch - 1)
    def _emit():
        l1 = acc_ref[0] / acc_ref[1]
        cd = acc_ref[2] * (1.0 / n_batch)
        out_ref[0, 0] = 0.5 * (l1 + cd)


@jax.jit
def kernel(pred, target, mask, points):
    B, N, D = pred.shape
    pad = [(0, 0), (0, 0), (0, 8 - D)]
    pred8 = jnp.pad(pred, pad)
    target8 = jnp.pad(target, pad)
    points8 = jnp.pad(points, pad)
    pred8T = jnp.transpose(pred8, (0, 2, 1))
    target8T = jnp.transpose(target8, (0, 2, 1))
    points8T = jnp.transpose(points8, (0, 2, 1))
    maskf = mask.astype(jnp.float32)
    maskT = maskf.reshape(B, 1, N)
    maskN = maskf.reshape(B, N, 1)

    out = pl.pallas_call(
        functools.partial(_chamfer_l1_kernel, n_batch=B),
        grid=(B,),
        in_specs=[
            pl.BlockSpec((1, 8, N), lambda b: (b, 0, 0)),
            pl.BlockSpec((1, 8, N), lambda b: (b, 0, 0)),
            pl.BlockSpec((1, 8, N), lambda b: (b, 0, 0)),
            pl.BlockSpec((1, 1, N), lambda b: (b, 0, 0)),
            pl.BlockSpec((1, N, 8), lambda b: (b, 0, 0)),
            pl.BlockSpec((1, N, 8), lambda b: (b, 0, 0)),
            pl.BlockSpec((1, N, 1), lambda b: (b, 0, 0)),
        ],
        out_specs=pl.BlockSpec((1, 1), lambda b: (0, 0),
                               memory_space=pltpu.SMEM),
        out_shape=jax.ShapeDtypeStruct((1, 1), jnp.float32),
        scratch_shapes=[pltpu.SMEM((4,), jnp.float32)],
    )(pred8T, target8T, points8T, maskT, target8, points8, maskN)
    return out[0, 0]
